# Initial kernel scaffold; baseline (speedup 1.0000x reference)
import jax
import jax.numpy as jnp
from jax.experimental import pallas as pl


def kernel(x, edge_index_rel1, edge_index_rel2, C_w, C_b, A_rel1, A_rel2):
    raise NotImplementedError("write your pallas kernel here")


# speedup vs baseline: 3.8277x; 3.8277x over previous
import functools

import jax
import jax.numpy as jnp
from jax import lax
from jax.experimental import pallas as pl
from jax.experimental.pallas import tpu as pltpu
from jax.experimental.pallas import tpu_sc as plsc

_N = 10000
_D = 128
_DA = 144
_ROWS = 10240
_W = 80
_TILES = 16
_RPT = _ROWS // _TILES


def _sc_body(xr, d1, s10, s11, d2, s20, s21, d3a, s3a, d3b, s3b,
             o10, o11, o20, o21, o22a, o22b,
             acc, stage, rows0, rows1, dstage, sstage, sem0, sem1):
    c = lax.axis_index("c")
    t = lax.axis_index("s")
    tbase = t * _RPT

    def _z_stage(i, _):
        stage[i // 9, pl.ds((i % 9) * 16, 16)] = jnp.zeros((16,), jnp.float32)
        return 0
    lax.fori_loop(0, 64 * 9, _z_stage, 0)

    def run_job(dref, sref, oref, K):
        for z in range(10):
            pltpu.sync_copy(stage, acc.at[pl.ds(tbase + z * 64, 64)])
        plsc.subcore_barrier()

        for ch in range(K // 16):
            co = t * K + ch * 16
            pltpu.sync_copy(dref.at[pl.ds(co, 16)], dstage)
            pltpu.sync_copy(sref.at[pl.ds(co, 16)], sstage)
            pltpu.async_copy(xr.at[sstage.at[0]], rows0, sem0)

            def step(k2, _):
                ka = 2 * k2
                kb = ka + 1
                kc = ka + 2
                pltpu.make_async_copy(xr.at[sstage.at[ka]], rows0, sem0).wait()
                pltpu.async_copy(xr.at[sstage.at[kb]], rows1, sem1)
                pltpu.sync_copy(rows0, acc.at[dstage.at[ka]], add=True)
                knext = jnp.minimum(kc, 15)

                @pl.when(kc < 16)
                def _():
                    pltpu.async_copy(xr.at[sstage.at[knext]], rows0, sem0)

                pltpu.make_async_copy(xr.at[sstage.at[kb]], rows1, sem1).wait()
                pltpu.sync_copy(rows1, acc.at[dstage.at[kb]], add=True)
                return 0
            lax.fori_loop(0, 8, step, 0)
        plsc.subcore_barrier()

        def fl(i, _):
            pltpu.sync_copy(acc.at[pl.ds(tbase + i * _W, _W)], rows0)
            pltpu.sync_copy(rows0, oref.at[pl.ds(tbase + i * _W, _W)])
            return 0
        lax.fori_loop(0, _RPT // _W, fl, 0)
        plsc.subcore_barrier()

    @pl.when(c == 0)
    def _():
        run_job(d1, s10, o10, 80)
        run_job(d2, s20, o20, 32)
        run_job(d3a, s3a, o22a, 16)

    @pl.when(c == 1)
    def _():
        run_job(d1, s11, o11, 80)
        run_job(d2, s21, o21, 32)
        run_job(d3b, s3b, o22b, 16)


def _sc_call(xa, d1, s10, s11, d2, s20, s21, d3a, s3a, d3b, s3b):
    f32 = jnp.float32
    big = jax.ShapeDtypeStruct((_ROWS, _DA), f32)
    mesh = plsc.VectorSubcoreMesh(core_axis_name="c", subcore_axis_name="s")
    fn = pl.kernel(
        _sc_body,
        out_type=[big] * 6,
        mesh=mesh,
        compiler_params=pltpu.CompilerParams(use_tc_tiling_on_sc=False),
        scratch_types=[
            pltpu.VMEM_SHARED((_ROWS, _DA), f32),
            pltpu.VMEM((64, _DA), f32),
            pltpu.VMEM((_W, _DA), f32),
            pltpu.VMEM((_W, _DA), f32),
            pltpu.VMEM((16, _W), jnp.int32),
            pltpu.VMEM((16, _W), jnp.int32),
            pltpu.SemaphoreType.DMA,
            pltpu.SemaphoreType.DMA,
        ],
    )
    return fn(xa, d1, s10, s11, d2, s20, s21, d3a, s3a, d3b, s3b)


def _tc_body(x_ref, p10, p11, p20, p21, p22a, p22b,
             a1a, a1b, a2a, a2b, a2c, cwt, cb, out_ref):
    f32 = jnp.float32
    dot = functools.partial(jnp.dot, preferred_element_type=f32)
    h = dot(x_ref[...], cwt[...]) + cb[...]
    n1 = p10[:, _D:_D + 1]
    r1 = jnp.where(n1 > 0, 1.0 / n1, 0.0)
    agg1 = dot(p10[:, :_D], a1a[...]) + dot(p11[:, :_D], a1b[...])
    n2 = p20[:, _D:_D + 1]
    r2 = jnp.where(n2 > 0, 1.0 / n2, 0.0)
    agg2 = (dot(p20[:, :_D], a2a[...]) + dot(p21[:, :_D], a2b[...])
            + dot(p22a[:, :_D] + p22b[:, :_D], a2c[...]))
    out_ref[...] = h + r1 * agg1 + r2 * agg2


def _tc_call(x, p10, p11, p20, p21, p22a, p22b,
             a1a, a1b, a2a, a2b, a2c, cwt, cb):
    blk = 1000
    row_spec = pl.BlockSpec((blk, _D), lambda i: (i, 0))
    p_spec = pl.BlockSpec((blk, _DA), lambda i: (i, 0))
    mat_spec = pl.BlockSpec((_D, _D), lambda i: (0, 0))
    bias_spec = pl.BlockSpec((1, _D), lambda i: (0, 0))
    return pl.pallas_call(
        _tc_body,
        grid=(_N // blk,),
        in_specs=[row_spec] + [p_spec] * 6 + [mat_spec] * 6 + [bias_spec],
        out_specs=row_spec,
        out_shape=jax.ShapeDtypeStruct((_N, _D), jnp.float32),
    )(x, p10, p11, p20, p21, p22a, p22b,
      a1a, a1b, a2a, a2b, a2c, cwt, cb)


def _pad_windows(dest, srcs, nwin):
    e = dest.shape[0]
    pad = nwin * _W - e
    i = jnp.arange(pad, dtype=jnp.int32)
    dpad = _N + (i % (_ROWS - _N))
    spad = (i * 997) % _N
    d = jnp.concatenate([dest, dpad]).reshape(nwin, _W)
    ss = [jnp.concatenate([s, spad]).reshape(nwin, _W) for s in srcs]
    return d, ss


def kernel(x, edge_index_rel1, edge_index_rel2, C_w, C_b, A_rel1, A_rel2):
    src1 = edge_index_rel1[0].reshape(-1, 2)
    dest1 = edge_index_rel1[1].reshape(-1, 2)[:, 0]
    src2 = edge_index_rel2[0].reshape(-1, 3)
    dest2 = edge_index_rel2[1].reshape(-1, 3)[:, 0]

    d1, (s10, s11) = _pad_windows(dest1, [src1[:, 0], src1[:, 1]], 1280)
    d2, (s20, s21) = _pad_windows(dest2, [src2[:, 0], src2[:, 1]], 512)
    half = dest2.shape[0] // 2
    s2c = src2[:, 2]
    d3a, (s3a,) = _pad_windows(dest2[:half], [s2c[:half]], 256)
    d3b, (s3b,) = _pad_windows(dest2[half:], [s2c[half:]], 256)

    xa = jnp.concatenate(
        [x, jnp.ones((_N, 1), jnp.float32), jnp.zeros((_N, _DA - _D - 1), jnp.float32)],
        axis=1)

    p10, p11, p20, p21, p22a, p22b = _sc_call(
        xa, d1, s10, s11, d2, s20, s21, d3a, s3a, d3b, s3b)

    a1a, a1b = A_rel1[:_D], A_rel1[_D:]
    a2a, a2b, a2c = A_rel2[:_D], A_rel2[_D:2 * _D], A_rel2[2 * _D:]
    return _tc_call(x, p10, p11, p20, p21, p22a, p22b,
                    a1a, a1b, a2a, a2b, a2c, C_w.T, C_b.reshape(1, _D))
